# baseline (speedup 1.0000x reference)
import jax
import jax.numpy as jnp
from jax.experimental import pallas as pl
from jax.experimental.pallas import tpu as pltpu
from jax.experimental.pallas import tpu_sc as plsc


def _transpose(X):
    B, L = X.shape
    blk = 512
    while B % blk:
        blk //= 2

    def body(x_ref, o_ref):
        o_ref[...] = x_ref[...].T

    return pl.pallas_call(
        body,
        grid=(B // blk,),
        in_specs=[pl.BlockSpec((blk, L), lambda i: (i, 0))],
        out_specs=pl.BlockSpec((L, blk), lambda i: (0, i)),
        out_shape=jax.ShapeDtypeStruct((L, B), X.dtype),
    )(X)


def _project_table(emb_table, Wx, b):
    V, D = emb_table.shape
    H = Wx.shape[1]
    blk = 4000
    while V % blk:
        blk //= 2
    b2 = b.reshape(1, H)

    def body(e_ref, wx_ref, b_ref, o_ref):
        acc = jnp.dot(
            e_ref[...].astype(jnp.bfloat16),
            wx_ref[...],
            preferred_element_type=jnp.float32,
        )
        o_ref[...] = acc + b_ref[...]

    return pl.pallas_call(
        body,
        grid=(V // blk,),
        in_specs=[
            pl.BlockSpec((blk, D), lambda i: (i, 0)),
            pl.BlockSpec((D, H), lambda i: (0, 0)),
            pl.BlockSpec((1, H), lambda i: (0, 0)),
        ],
        out_specs=pl.BlockSpec((blk, H), lambda i: (i, 0)),
        out_shape=jax.ShapeDtypeStruct((V, H), jnp.float32),
    )(emb_table, Wx.astype(jnp.bfloat16), b2)


def _sc_gather(P, idx, window=128):
    Lc, B = idx.shape
    N = Lc * B
    H = P.shape[1]
    wpr = B // window
    mesh = plsc.VectorSubcoreMesh(core_axis_name="core", subcore_axis_name="subcore")

    @pl.kernel(
        out_type=jax.ShapeDtypeStruct((N, H), P.dtype),
        mesh=mesh,
    )
    def k(p_hbm, i_hbm, o_hbm):
        def body(i_vmem, o_vmem):
            pltpu.sync_copy(p_hbm.at[i_vmem.at[0]], o_vmem)

        pltpu.emit_pipeline(
            body,
            grid=(N // window,),
            in_specs=[
                pl.BlockSpec((1, window), index_map=lambda i: (i // wpr, i % wpr))
            ],
            out_specs=[pl.BlockSpec((window, H), index_map=lambda i: (i, 0))],
            core_axis_name=("core", "subcore"),
            dimension_semantics=(pltpu.PARALLEL,),
        )(i_hbm, o_hbm)

    return k(P, idx)


def _rnn_chunk(U, h0, Wh16, ms, first, Wd=None, bd2=None):
    Lc, B, H = U.shape
    last = Wd is not None
    grid = Lc // ms

    def body(u_ref, h0_ref, wh_ref, wd_ref, bd_ref, o_ref, h_ref):
        t = pl.program_id(0)

        def mm(h):
            return jnp.dot(
                h.astype(jnp.bfloat16), wh_ref[...],
                preferred_element_type=jnp.float32,
            )

        @pl.when(t == 0)
        def _():
            if first:
                h_ref[...] = jnp.tanh(u_ref[0])
            else:
                h_ref[...] = jnp.tanh(u_ref[0] + mm(h0_ref[...]))

        @pl.when(t > 0)
        def _():
            h_ref[...] = jnp.tanh(u_ref[0] + mm(h_ref[...]))

        for j in range(1, ms):
            h_ref[...] = jnp.tanh(u_ref[j] + mm(h_ref[...]))

        @pl.when(t == grid - 1)
        def _():
            if last:
                logits = (
                    jnp.dot(
                        h_ref[...], wd_ref[...], preferred_element_type=jnp.float32
                    )
                    + bd_ref[...]
                )
                o_ref[...] = jax.nn.sigmoid(logits)
            else:
                o_ref[...] = h_ref[...]

    if not last:
        Wd = jnp.zeros((H, 1), dtype=jnp.float32)
        bd2 = jnp.zeros((1, 1), dtype=jnp.float32)
    out_shape = (B, 1) if last else (B, H)
    return pl.pallas_call(
        body,
        grid=(grid,),
        in_specs=[
            pl.BlockSpec((ms, B, H), lambda t: (t, 0, 0)),
            pl.BlockSpec((B, H), lambda t: (0, 0)),
            pl.BlockSpec((H, H), lambda t: (0, 0)),
            pl.BlockSpec((H, 1), lambda t: (0, 0)),
            pl.BlockSpec((1, 1), lambda t: (0, 0)),
        ],
        out_specs=pl.BlockSpec(out_shape, lambda t: (0, 0)),
        out_shape=jax.ShapeDtypeStruct(out_shape, jnp.float32),
        scratch_shapes=[pltpu.VMEM((B, H), jnp.float32)],
    )(U, h0, Wh16, Wd, bd2)


_NCHUNKS = 5
_MS = 5


def kernel(X, emb_table, Wx, Wh, b, Wd, bd):
    B, L = X.shape
    H = Wh.shape[0]
    nchunks = _NCHUNKS
    while L % nchunks:
        nchunks -= 1
    Lc = L // nchunks
    ms = _MS
    while Lc % ms:
        ms -= 1

    Xt = _transpose(X.astype(jnp.int32))
    P = _project_table(emb_table, Wx, b)
    Wh16 = Wh.astype(jnp.bfloat16)
    bd2 = bd.reshape(1, 1)

    Us = []
    for c in range(nchunks):
        idx = jax.lax.slice(Xt, (c * Lc, 0), ((c + 1) * Lc, B))
        Us.append(_sc_gather(P, idx).reshape(Lc, B, H))

    h = jnp.zeros((B, H), dtype=jnp.float32)
    for c in range(nchunks):
        is_last = c == nchunks - 1
        h = _rnn_chunk(
            Us[c],
            h,
            Wh16,
            ms,
            first=(c == 0),
            Wd=Wd if is_last else None,
            bd2=bd2 if is_last else None,
        )
    return h


# speedup vs baseline: 1.0192x; 1.0192x over previous
import jax
import jax.numpy as jnp
from jax.experimental import pallas as pl
from jax.experimental.pallas import tpu as pltpu
from jax.experimental.pallas import tpu_sc as plsc


def _transpose(X):
    B, L = X.shape
    blk = 512
    while B % blk:
        blk //= 2

    def body(x_ref, o_ref):
        o_ref[...] = x_ref[...].T

    return pl.pallas_call(
        body,
        grid=(B // blk,),
        in_specs=[pl.BlockSpec((blk, L), lambda i: (i, 0))],
        out_specs=pl.BlockSpec((L, blk), lambda i: (0, i)),
        out_shape=jax.ShapeDtypeStruct((L, B), X.dtype),
    )(X)


def _project_table(emb_table, Wx, b):
    V, D = emb_table.shape
    H = Wx.shape[1]
    blk = 4000
    while V % blk:
        blk //= 2
    b2 = b.reshape(1, H)

    def body(e_ref, wx_ref, b_ref, o_ref):
        acc = jnp.dot(
            e_ref[...].astype(jnp.bfloat16),
            wx_ref[...],
            preferred_element_type=jnp.float32,
        )
        o_ref[...] = acc + b_ref[...]

    return pl.pallas_call(
        body,
        grid=(V // blk,),
        in_specs=[
            pl.BlockSpec((blk, D), lambda i: (i, 0)),
            pl.BlockSpec((D, H), lambda i: (0, 0)),
            pl.BlockSpec((1, H), lambda i: (0, 0)),
        ],
        out_specs=pl.BlockSpec((blk, H), lambda i: (i, 0)),
        out_shape=jax.ShapeDtypeStruct((V, H), jnp.float32),
    )(emb_table, Wx.astype(jnp.bfloat16), b2)


def _sc_gather(P, idx, window=128):
    Lc, B = idx.shape
    N = Lc * B
    H = P.shape[1]
    wpr = B // window
    mesh = plsc.VectorSubcoreMesh(core_axis_name="core", subcore_axis_name="subcore")

    @pl.kernel(
        out_type=jax.ShapeDtypeStruct((N, H), P.dtype),
        mesh=mesh,
    )
    def k(p_hbm, i_hbm, o_hbm):
        def body(i_vmem, o_vmem):
            pltpu.sync_copy(p_hbm.at[i_vmem.at[0]], o_vmem)

        pltpu.emit_pipeline(
            body,
            grid=(N // window,),
            in_specs=[
                pl.BlockSpec((1, window), index_map=lambda i: (i // wpr, i % wpr))
            ],
            out_specs=[pl.BlockSpec((window, H), index_map=lambda i: (i, 0))],
            core_axis_name=("core", "subcore"),
            dimension_semantics=(pltpu.PARALLEL,),
        )(i_hbm, o_hbm)

    return k(P, idx)


def _rnn_chunk(U, h0, Wh16, ms, first, Wd=None, bd2=None):
    Lc, B, H = U.shape
    last = Wd is not None
    grid = Lc // ms

    def body(u_ref, h0_ref, wh_ref, wd_ref, bd_ref, o_ref, h_ref):
        t = pl.program_id(0)

        def mm(h):
            return jnp.dot(
                h.astype(jnp.bfloat16), wh_ref[...],
                preferred_element_type=jnp.float32,
            )

        @pl.when(t == 0)
        def _():
            if first:
                h_ref[...] = jnp.tanh(u_ref[0])
            else:
                h_ref[...] = jnp.tanh(u_ref[0] + mm(h0_ref[...]))

        @pl.when(t > 0)
        def _():
            h_ref[...] = jnp.tanh(u_ref[0] + mm(h_ref[...]))

        for j in range(1, ms):
            h_ref[...] = jnp.tanh(u_ref[j] + mm(h_ref[...]))

        @pl.when(t == grid - 1)
        def _():
            if last:
                logits = (
                    jnp.dot(
                        h_ref[...], wd_ref[...], preferred_element_type=jnp.float32
                    )
                    + bd_ref[...]
                )
                o_ref[...] = jax.nn.sigmoid(logits)
            else:
                o_ref[...] = h_ref[...]

    if not last:
        Wd = jnp.zeros((H, 1), dtype=jnp.float32)
        bd2 = jnp.zeros((1, 1), dtype=jnp.float32)
    out_shape = (B, 1) if last else (B, H)
    return pl.pallas_call(
        body,
        grid=(grid,),
        in_specs=[
            pl.BlockSpec((ms, B, H), lambda t: (t, 0, 0)),
            pl.BlockSpec((B, H), lambda t: (0, 0)),
            pl.BlockSpec((H, H), lambda t: (0, 0)),
            pl.BlockSpec((H, 1), lambda t: (0, 0)),
            pl.BlockSpec((1, 1), lambda t: (0, 0)),
        ],
        out_specs=pl.BlockSpec(out_shape, lambda t: (0, 0)),
        out_shape=jax.ShapeDtypeStruct(out_shape, jnp.float32),
        scratch_shapes=[pltpu.VMEM((B, H), jnp.float32)],
    )(U, h0, Wh16, Wd, bd2)


_NCHUNKS = 5
_MS = 5


def kernel(X, emb_table, Wx, Wh, b, Wd, bd):
    B, L = X.shape
    H = Wh.shape[0]
    nchunks = _NCHUNKS
    while L % nchunks:
        nchunks -= 1
    Lc = L // nchunks
    ms = _MS
    while Lc % ms:
        ms -= 1

    Xt = X.astype(jnp.int32).T
    P = _project_table(emb_table, Wx, b)
    Wh16 = Wh.astype(jnp.bfloat16)
    bd2 = bd.reshape(1, 1)

    Us = []
    for c in range(nchunks):
        idx = jax.lax.slice(Xt, (c * Lc, 0), ((c + 1) * Lc, B))
        Us.append(_sc_gather(P, idx).reshape(Lc, B, H))

    h = jnp.zeros((B, H), dtype=jnp.float32)
    for c in range(nchunks):
        is_last = c == nchunks - 1
        h = _rnn_chunk(
            Us[c],
            h,
            Wh16,
            ms,
            first=(c == 0),
            Wd=Wd if is_last else None,
            bd2=bd2 if is_last else None,
        )
    return h


# speedup vs baseline: 1.0972x; 1.0765x over previous
import jax
import jax.numpy as jnp
from jax.experimental import pallas as pl
from jax.experimental.pallas import tpu as pltpu
from jax.experimental.pallas import tpu_sc as plsc


def _transpose(X):
    B, L = X.shape
    blk = 512
    while B % blk:
        blk //= 2

    def body(x_ref, o_ref):
        o_ref[...] = x_ref[...].T

    return pl.pallas_call(
        body,
        grid=(B // blk,),
        in_specs=[pl.BlockSpec((blk, L), lambda i: (i, 0))],
        out_specs=pl.BlockSpec((L, blk), lambda i: (0, i)),
        out_shape=jax.ShapeDtypeStruct((L, B), X.dtype),
    )(X)


def _project_table(emb_table, Wx, b):
    V, D = emb_table.shape
    H = Wx.shape[1]
    blk = 4000
    while V % blk:
        blk //= 2
    b2 = b.reshape(1, H)

    def body(e_ref, wx_ref, b_ref, o_ref):
        acc = jnp.dot(
            e_ref[...].astype(jnp.bfloat16),
            wx_ref[...],
            preferred_element_type=jnp.float32,
        )
        o_ref[...] = acc + b_ref[...]

    return pl.pallas_call(
        body,
        grid=(V // blk,),
        in_specs=[
            pl.BlockSpec((blk, D), lambda i: (i, 0)),
            pl.BlockSpec((D, H), lambda i: (0, 0)),
            pl.BlockSpec((1, H), lambda i: (0, 0)),
        ],
        out_specs=pl.BlockSpec((blk, H), lambda i: (i, 0)),
        out_shape=jax.ShapeDtypeStruct((V, H), jnp.float32),
    )(emb_table, Wx.astype(jnp.bfloat16), b2)


def _sc_gather(P, idx, window=256):
    Lc, B = idx.shape
    N = Lc * B
    H = P.shape[1]
    wpr = B // window
    mesh = plsc.VectorSubcoreMesh(core_axis_name="core", subcore_axis_name="subcore")

    @pl.kernel(
        out_type=jax.ShapeDtypeStruct((N, H), P.dtype),
        mesh=mesh,
    )
    def k(p_hbm, i_hbm, o_hbm):
        def body(i_vmem, o_vmem):
            pltpu.sync_copy(p_hbm.at[i_vmem.at[0]], o_vmem)

        pltpu.emit_pipeline(
            body,
            grid=(N // window,),
            in_specs=[
                pl.BlockSpec((1, window), index_map=lambda i: (i // wpr, i % wpr))
            ],
            out_specs=[pl.BlockSpec((window, H), index_map=lambda i: (i, 0))],
            core_axis_name=("core", "subcore"),
            dimension_semantics=(pltpu.PARALLEL,),
        )(i_hbm, o_hbm)

    return k(P, idx)


def _rnn_chunk(U, h0, Wh16, ms, first, Wd=None, bd2=None):
    Lc, B, H = U.shape
    last = Wd is not None
    grid = Lc // ms

    def body(u_ref, h0_ref, wh_ref, wd_ref, bd_ref, o_ref, h_ref):
        t = pl.program_id(0)

        def mm(h):
            return jnp.dot(
                h.astype(jnp.bfloat16), wh_ref[...],
                preferred_element_type=jnp.float32,
            )

        @pl.when(t == 0)
        def _():
            if first:
                h_ref[...] = jnp.tanh(u_ref[0])
            else:
                h_ref[...] = jnp.tanh(u_ref[0] + mm(h0_ref[...]))

        @pl.when(t > 0)
        def _():
            h_ref[...] = jnp.tanh(u_ref[0] + mm(h_ref[...]))

        for j in range(1, ms):
            h_ref[...] = jnp.tanh(u_ref[j] + mm(h_ref[...]))

        @pl.when(t == grid - 1)
        def _():
            if last:
                logits = (
                    jnp.dot(
                        h_ref[...], wd_ref[...], preferred_element_type=jnp.float32
                    )
                    + bd_ref[...]
                )
                o_ref[...] = jax.nn.sigmoid(logits)
            else:
                o_ref[...] = h_ref[...]

    if not last:
        Wd = jnp.zeros((H, 1), dtype=jnp.float32)
        bd2 = jnp.zeros((1, 1), dtype=jnp.float32)
    out_shape = (B, 1) if last else (B, H)
    return pl.pallas_call(
        body,
        grid=(grid,),
        in_specs=[
            pl.BlockSpec((ms, B, H), lambda t: (t, 0, 0)),
            pl.BlockSpec((B, H), lambda t: (0, 0)),
            pl.BlockSpec((H, H), lambda t: (0, 0)),
            pl.BlockSpec((H, 1), lambda t: (0, 0)),
            pl.BlockSpec((1, 1), lambda t: (0, 0)),
        ],
        out_specs=pl.BlockSpec(out_shape, lambda t: (0, 0)),
        out_shape=jax.ShapeDtypeStruct(out_shape, jnp.float32),
        scratch_shapes=[pltpu.VMEM((B, H), jnp.float32)],
    )(U, h0, Wh16, Wd, bd2)


_NCHUNKS = 5
_MS = 5


def kernel(X, emb_table, Wx, Wh, b, Wd, bd):
    B, L = X.shape
    H = Wh.shape[0]
    nchunks = _NCHUNKS
    while L % nchunks:
        nchunks -= 1
    Lc = L // nchunks
    ms = _MS
    while Lc % ms:
        ms -= 1

    Xt = X.astype(jnp.int32).T
    P = _project_table(emb_table, Wx, b)
    Wh16 = Wh.astype(jnp.bfloat16)
    bd2 = bd.reshape(1, 1)

    Us = []
    for c in range(nchunks):
        idx = jax.lax.slice(Xt, (c * Lc, 0), ((c + 1) * Lc, B))
        Us.append(_sc_gather(P, idx).reshape(Lc, B, H))

    h = jnp.zeros((B, H), dtype=jnp.float32)
    for c in range(nchunks):
        is_last = c == nchunks - 1
        h = _rnn_chunk(
            Us[c],
            h,
            Wh16,
            ms,
            first=(c == 0),
            Wd=Wd if is_last else None,
            bd2=bd2 if is_last else None,
        )
    return h


# speedup vs baseline: 1.1044x; 1.0065x over previous
import jax
import jax.numpy as jnp
from jax.experimental import pallas as pl
from jax.experimental.pallas import tpu as pltpu
from jax.experimental.pallas import tpu_sc as plsc


def _transpose(X):
    B, L = X.shape
    blk = 512
    while B % blk:
        blk //= 2

    def body(x_ref, o_ref):
        o_ref[...] = x_ref[...].T

    return pl.pallas_call(
        body,
        grid=(B // blk,),
        in_specs=[pl.BlockSpec((blk, L), lambda i: (i, 0))],
        out_specs=pl.BlockSpec((L, blk), lambda i: (0, i)),
        out_shape=jax.ShapeDtypeStruct((L, B), X.dtype),
    )(X)


def _project_table(emb_table, Wx, b):
    V, D = emb_table.shape
    H = Wx.shape[1]
    blk = 4000
    while V % blk:
        blk //= 2
    b2 = b.reshape(1, H)

    def body(e_ref, wx_ref, b_ref, o_ref):
        acc = jnp.dot(
            e_ref[...].astype(jnp.bfloat16),
            wx_ref[...],
            preferred_element_type=jnp.float32,
        )
        o_ref[...] = acc + b_ref[...]

    return pl.pallas_call(
        body,
        grid=(V // blk,),
        in_specs=[
            pl.BlockSpec((blk, D), lambda i: (i, 0)),
            pl.BlockSpec((D, H), lambda i: (0, 0)),
            pl.BlockSpec((1, H), lambda i: (0, 0)),
        ],
        out_specs=pl.BlockSpec((blk, H), lambda i: (i, 0)),
        out_shape=jax.ShapeDtypeStruct((V, H), jnp.float32),
    )(emb_table, Wx.astype(jnp.bfloat16), b2)


def _sc_gather(P, idx, window=256):
    Lc, B = idx.shape
    N = Lc * B
    H = P.shape[1]
    wpr = B // window
    mesh = plsc.VectorSubcoreMesh(core_axis_name="core", subcore_axis_name="subcore")

    @pl.kernel(
        out_type=jax.ShapeDtypeStruct((N, H), P.dtype),
        mesh=mesh,
    )
    def k(p_hbm, i_hbm, o_hbm):
        def body(i_vmem, o_vmem):
            pltpu.sync_copy(p_hbm.at[i_vmem.at[0]], o_vmem)

        pltpu.emit_pipeline(
            body,
            grid=(N // window,),
            in_specs=[
                pl.BlockSpec((1, window), index_map=lambda i: (i // wpr, i % wpr))
            ],
            out_specs=[pl.BlockSpec((window, H), index_map=lambda i: (i, 0))],
            core_axis_name=("core", "subcore"),
            dimension_semantics=(pltpu.PARALLEL,),
        )(i_hbm, o_hbm)

    return k(P, idx)


def _rnn_chunk(U, h0, Wh16, ms, first, Wd=None, bd2=None):
    Lc, B, H = U.shape
    last = Wd is not None
    grid = Lc // ms

    def body(u_ref, h0_ref, wh_ref, wd_ref, bd_ref, o_ref, h_ref):
        t = pl.program_id(0)

        def mm(h):
            return jnp.dot(
                h.astype(jnp.bfloat16), wh_ref[...],
                preferred_element_type=jnp.float32,
            )

        @pl.when(t == 0)
        def _():
            if first:
                h_ref[...] = jnp.tanh(u_ref[0])
            else:
                h_ref[...] = jnp.tanh(u_ref[0] + mm(h0_ref[...]))

        @pl.when(t > 0)
        def _():
            h_ref[...] = jnp.tanh(u_ref[0] + mm(h_ref[...]))

        for j in range(1, ms):
            h_ref[...] = jnp.tanh(u_ref[j] + mm(h_ref[...]))

        @pl.when(t == grid - 1)
        def _():
            if last:
                logits = (
                    jnp.dot(
                        h_ref[...], wd_ref[...], preferred_element_type=jnp.float32
                    )
                    + bd_ref[...]
                )
                o_ref[...] = jax.nn.sigmoid(logits)
            else:
                o_ref[...] = h_ref[...]

    if not last:
        Wd = jnp.zeros((H, 1), dtype=jnp.float32)
        bd2 = jnp.zeros((1, 1), dtype=jnp.float32)
    out_shape = (B, 1) if last else (B, H)
    return pl.pallas_call(
        body,
        grid=(grid,),
        in_specs=[
            pl.BlockSpec((ms, B, H), lambda t: (t, 0, 0)),
            pl.BlockSpec((B, H), lambda t: (0, 0)),
            pl.BlockSpec((H, H), lambda t: (0, 0)),
            pl.BlockSpec((H, 1), lambda t: (0, 0)),
            pl.BlockSpec((1, 1), lambda t: (0, 0)),
        ],
        out_specs=pl.BlockSpec(out_shape, lambda t: (0, 0)),
        out_shape=jax.ShapeDtypeStruct(out_shape, jnp.float32),
        scratch_shapes=[pltpu.VMEM((B, H), jnp.float32)],
    )(U, h0, Wh16, Wd, bd2)


_NCHUNKS = 4
_MS = 5


def kernel(X, emb_table, Wx, Wh, b, Wd, bd):
    B, L = X.shape
    H = Wh.shape[0]
    nchunks = _NCHUNKS
    while L % nchunks:
        nchunks -= 1
    Lc = L // nchunks
    ms = _MS
    while Lc % ms:
        ms -= 1

    Xt = X.astype(jnp.int32).T
    P = _project_table(emb_table, Wx, b)
    Wh16 = Wh.astype(jnp.bfloat16)
    bd2 = bd.reshape(1, 1)

    Us = []
    for c in range(nchunks):
        idx = jax.lax.slice(Xt, (c * Lc, 0), ((c + 1) * Lc, B))
        Us.append(_sc_gather(P, idx).reshape(Lc, B, H))

    h = jnp.zeros((B, H), dtype=jnp.float32)
    for c in range(nchunks):
        is_last = c == nchunks - 1
        h = _rnn_chunk(
            Us[c],
            h,
            Wh16,
            ms,
            first=(c == 0),
            Wd=Wd if is_last else None,
            bd2=bd2 if is_last else None,
        )
    return h
